# baseline (speedup 1.0000x reference)
import functools

import jax
import jax.numpy as jnp
from jax import lax
from jax.experimental import pallas as pl
from jax.experimental.pallas import tpu as pltpu
from jax.experimental.pallas import tpu_sc as plsc


@functools.cache
def _make_gather(V, D, B):
    info = plsc.get_sparse_core_info()
    NC, NS, L = info.num_cores, info.num_subcores, info.num_lanes
    assert B % (NC * NS * L) == 0 and D == 2 * L
    half = B // NC
    outw = half // NS
    CHUNK = 1024
    NCH = (V + CHUNK - 1) // CHUNK
    TAILW = (V - (NCH - 1) * CHUNK + 127) // 128 * 128
    KMAX = (NCH + NS - 1) // NS
    mesh = plsc.VectorSubcoreMesh(core_axis_name="c", subcore_axis_name="s")

    @functools.partial(
        pl.kernel,
        mesh=mesh,
        compiler_params=pltpu.CompilerParams(
            disable_bounds_checks=True,
            disable_semaphore_checks=True,
            needs_layout_passes=False,
        ),
        out_type=jax.ShapeDtypeStruct((D, B), jnp.float32),
        scratch_types=[
            pltpu.VMEM((half,), jnp.int32),
            pltpu.VMEM((half + L,), jnp.int32),
            pltpu.VMEM((half + L,), jnp.int32),
            pltpu.VMEM((half + L,), jnp.int32),
            pltpu.VMEM((half + L,), jnp.int32),
            pltpu.VMEM((D, CHUNK), jnp.float32),
            pltpu.VMEM((D, CHUNK), jnp.float32),
            pltpu.VMEM((2 * L * D,), jnp.float32),
            pltpu.VMEM((outw * D,), jnp.float32),
            pltpu.VMEM((D, outw), jnp.float32),
            pltpu.VMEM((D,), jnp.int32),
            pltpu.VMEM_SHARED((half * D,), jnp.float32),
            pltpu.SemaphoreType.DMA,
            pltpu.SemaphoreType.DMA,
            pltpu.SemaphoreType.DMA,
        ],
    )
    def gather_kernel(t_hbm, idx_hbm, out_t_hbm,
                      idx_v, wl_c, wl_b, sub_c, sub_b, chunk_a, chunk_b,
                      own_r, wbsrc, wb2d, drain_v, spm, sem, sem_a, sem_b):
        cid = lax.axis_index("c")
        tid = lax.axis_index("s")
        iota = lax.iota(jnp.int32, L)

        bufs = [chunk_a, chunk_b]
        sems = [sem_a, sem_b]


        def filt_make(gch):
            def filt(g, m):
                c = wl_c[pl.ds(g * L, L)]
                b = wl_b[pl.ds(g * L, L)]
                lanepos = g * L + iota
                m2 = ((c >> 10) == gch) & (lanepos < n)
                pos = m + plsc.cumsum(m2.astype(jnp.int32)) - 1
                plsc.store_scatter(sub_c, [pos], c, mask=m2)
                plsc.store_scatter(sub_b, [pos], b, mask=m2)
                return pos[L - 1] + 1
            return filt

        def pub_drain(cnt):
            def drain(i, _):
                pltpu.make_async_copy(
                    idx_hbm.at[pl.ds(0, D)], drain_v, sem,
                ).wait()
                return 0

            lax.fori_loop(0, cnt, drain, 0)

        def proc_chunk(buf, gch, m):
            base = gch * CHUNK
            nfull = m // L

            def emit(g, slot_off, l, guard):
                cs = sub_c[pl.ds(g * L, L)]
                bs = sub_b[pl.ds(g * L, L)]

                def one():
                    cl = jnp.full((L,), cs[l] - base, jnp.int32)
                    v0 = plsc.load_gather(buf, [iota, cl])
                    v1 = plsc.load_gather(buf, [iota + L, cl])
                    own_r[pl.ds(slot_off + l * D, L)] = v0
                    own_r[pl.ds(slot_off + l * D + L, L)] = v1
                    pltpu.make_async_copy(
                        own_r.at[pl.ds(slot_off + l * D, D)],
                        spm.at[pl.ds(bs[l] * D, D)],
                        sem,
                    ).start()

                if guard:
                    pl.when(g * L + l < m)(one)
                else:
                    one()

            def proc(g, prev):
                slot_off = lax.rem(g, 2) * (L * D)
                for l in range(L):
                    emit(g, slot_off, l, guard=False)
                pub_drain(prev)
                return jnp.int32(L)

            prev = lax.fori_loop(0, nfull, proc, jnp.int32(0))
            tail = m - nfull * L

            @pl.when(tail > 0)
            def _():
                slot_off = lax.rem(nfull, 2) * (L * D)
                for l in range(L):
                    emit(nfull, slot_off, l, guard=True)

            pub_drain(prev + tail)

        def dfetch(kd):
            gch = kd * NS + tid
            for par in range(2):
                @pl.when((lax.rem(kd, 2) == par) & (gch < NCH - 1))
                def _(par=par, gch=gch):
                    pltpu.make_async_copy(
                        t_hbm.at[:, pl.ds(pl.multiple_of(gch * CHUNK, 128),
                                          CHUNK)],
                        bufs[par], sems[par]).start()

                @pl.when((lax.rem(kd, 2) == par) & (gch == NCH - 1))
                def _(par=par, gch=gch):
                    pltpu.make_async_copy(
                        t_hbm.at[:, pl.ds(pl.multiple_of(gch * CHUNK, 128),
                                          TAILW)],
                        bufs[par].at[:, pl.ds(0, TAILW)], sems[par]).start()

        def dwait(kd):
            gch = kd * NS + tid
            for par in range(2):
                @pl.when((lax.rem(kd, 2) == par) & (gch < NCH - 1))
                def _(par=par):
                    pltpu.make_async_copy(
                        t_hbm.at[:, pl.ds(0, CHUNK)], bufs[par],
                        sems[par]).wait()

                @pl.when((lax.rem(kd, 2) == par) & (gch == NCH - 1))
                def _(par=par):
                    pltpu.make_async_copy(
                        t_hbm.at[:, pl.ds(0, TAILW)],
                        bufs[par].at[:, pl.ds(0, TAILW)], sems[par]).wait()

        def do_chunk(kd, _):
            gch = kd * NS + tid

            @pl.when(gch <= NCH - 1)
            def _():
                m = lax.fori_loop(0, nv, filt_make(gch), jnp.int32(0))
                dwait(kd)
                for par in range(2):
                    @pl.when((lax.rem(kd, 2) == par) & (m > 0))
                    def _(par=par):
                        proc_chunk(bufs[par], gch, m)

            @pl.when(kd + 2 <= KMAX - 1)
            def _():
                dfetch(kd + 2)
            return 0

        dfetch(jnp.int32(0))
        if KMAX > 1:
            dfetch(jnp.int32(1))

        pltpu.sync_copy(idx_hbm.at[pl.ds(cid * half, half)], idx_v)

        def buck(j, n):
            vec = idx_v[pl.ds(j * L, L)]
            mask = ((vec >> 10) & (NS - 1)) == tid
            pos = n + plsc.cumsum(mask.astype(jnp.int32)) - 1
            plsc.store_scatter(wl_c, [pos], vec, mask=mask)
            bpos = j * L + iota
            plsc.store_scatter(wl_b, [pos], bpos, mask=mask)
            return pos[L - 1] + 1

        n = lax.fori_loop(0, half // L, buck, jnp.int32(0))
        nv = (n + L - 1) // L

        lax.fori_loop(0, KMAX, do_chunk, 0)

        plsc.subcore_barrier()
        pltpu.sync_copy(spm.at[pl.ds(tid * (outw * D), outw * D)], wbsrc)
        iotaD = iota * D

        def transpose_row(d, _):
            for g in range(outw // L):
                val = plsc.load_gather(wbsrc, [iotaD + (g * L * D + d)])
                wb2d[d, pl.ds(g * L, L)] = val
            return 0

        lax.fori_loop(0, D, transpose_row, 0)
        pltpu.sync_copy(
            wb2d,
            out_t_hbm.at[:, pl.ds(pl.multiple_of(cid * half + tid * outw, 128),
                                  outw)])

    return gather_kernel


def kernel(user_id, embedding_table):
    (B,) = user_id.shape
    V, D = embedding_table.shape
    idx = user_id.astype(jnp.int32)
    out_t = _make_gather(V, D, B)(embedding_table.T, idx)
    return out_t.T


# speedup vs baseline: 1.0446x; 1.0446x over previous
import functools

import jax
import jax.numpy as jnp
from jax import lax
from jax.experimental import pallas as pl
from jax.experimental.pallas import tpu as pltpu
from jax.experimental.pallas import tpu_sc as plsc


@functools.cache
def _make_gather(V, D, B):
    info = plsc.get_sparse_core_info()
    NC, NS, L = info.num_cores, info.num_subcores, info.num_lanes
    assert B % (NC * NS * L) == 0 and D == 2 * L
    half = B // NC
    outw = half // NS
    CHUNK = 1024
    NCH = (V + CHUNK - 1) // CHUNK
    TAILW = (V - (NCH - 1) * CHUNK + 127) // 128 * 128
    KMAX = (NCH + NS - 1) // NS
    mesh = plsc.VectorSubcoreMesh(core_axis_name="c", subcore_axis_name="s")

    @functools.partial(
        pl.kernel,
        mesh=mesh,
        compiler_params=pltpu.CompilerParams(
            disable_bounds_checks=True,
            disable_semaphore_checks=True,
            needs_layout_passes=False,
        ),
        out_type=jax.ShapeDtypeStruct((D, B), jnp.float32),
        scratch_types=[
            pltpu.VMEM((half,), jnp.int32),
            pltpu.VMEM((half + L,), jnp.int32),
            pltpu.VMEM((half + L,), jnp.int32),
            pltpu.VMEM((half + L,), jnp.int32),
            pltpu.VMEM((half + L,), jnp.int32),
            pltpu.VMEM((D, CHUNK), jnp.float32),
            pltpu.VMEM((D, CHUNK), jnp.float32),
            pltpu.VMEM((2 * L * D,), jnp.float32),
            pltpu.VMEM((outw * D,), jnp.float32),
            pltpu.VMEM((D, outw), jnp.float32),
            pltpu.VMEM((D,), jnp.int32),
            pltpu.VMEM_SHARED((half * D,), jnp.float32),
            pltpu.SemaphoreType.DMA,
            pltpu.SemaphoreType.DMA,
            pltpu.SemaphoreType.DMA,
        ],
    )
    def gather_kernel(t_hbm, idx_hbm, out_t_hbm,
                      idx_v, wl_c, wl_b, sub_c, sub_b, chunk_a, chunk_b,
                      own_r, wbsrc, wb2d, drain_v, spm, sem, sem_a, sem_b):
        cid = lax.axis_index("c")
        tid = lax.axis_index("s")
        iota = lax.iota(jnp.int32, L)

        bufs = [chunk_a, chunk_b]
        sems = [sem_a, sem_b]


        def filt_make(gch):
            def filt(g, m):
                c = wl_c[pl.ds(g * L, L)]
                b = wl_b[pl.ds(g * L, L)]
                lanepos = g * L + iota
                m2 = ((c >> 10) == gch) & (lanepos < n)
                pos = m + plsc.cumsum(m2.astype(jnp.int32)) - 1
                plsc.store_scatter(sub_c, [pos], c, mask=m2)
                plsc.store_scatter(sub_b, [pos], b, mask=m2)
                return pos[L - 1] + 1
            return filt

        def pub_drain(cnt):
            def drain(i, _):
                pltpu.make_async_copy(
                    idx_hbm.at[pl.ds(0, D)], drain_v, sem,
                ).wait()
                return 0

            lax.fori_loop(0, cnt, drain, 0)

        def proc_chunk(buf, gch, m):
            base = gch * CHUNK
            nfull = m // L

            def emit(cs, bs, slot_off, l, pred):

                def one():
                    cl = jnp.full((L,), cs[l] - base, jnp.int32)
                    v0 = plsc.load_gather(buf, [iota, cl])
                    v1 = plsc.load_gather(buf, [iota + L, cl])
                    own_r[pl.ds(slot_off + l * D, L)] = v0
                    own_r[pl.ds(slot_off + l * D + L, L)] = v1
                    pltpu.make_async_copy(
                        own_r.at[pl.ds(slot_off + l * D, D)],
                        spm.at[pl.ds(bs[l] * D, D)],
                        sem,
                    ).start()

                if pred is None:
                    one()
                else:
                    pl.when(pred)(one)

            def proc(g, prev):
                slot_off = lax.rem(g, 2) * (L * D)
                cs = sub_c[pl.ds(g * L, L)]
                bs = sub_b[pl.ds(g * L, L)]
                for l in range(L):
                    emit(cs, bs, slot_off, l, None)
                pub_drain(prev)
                return jnp.int32(L)

            prev = lax.fori_loop(0, nfull, proc, jnp.int32(0))
            tail = m - nfull * L

            @pl.when(tail > 0)
            def _():
                slot_off = lax.rem(nfull, 2) * (L * D)
                cs = sub_c[pl.ds(nfull * L, L)]
                bs = sub_b[pl.ds(nfull * L, L)]
                for l in range(L):
                    emit(cs, bs, slot_off, l, nfull * L + l < m)

            pub_drain(prev + tail)

        def dfetch(kd):
            gch = kd * NS + tid
            for par in range(2):
                @pl.when((lax.rem(kd, 2) == par) & (gch < NCH - 1))
                def _(par=par, gch=gch):
                    pltpu.make_async_copy(
                        t_hbm.at[:, pl.ds(pl.multiple_of(gch * CHUNK, 128),
                                          CHUNK)],
                        bufs[par], sems[par]).start()

                @pl.when((lax.rem(kd, 2) == par) & (gch == NCH - 1))
                def _(par=par, gch=gch):
                    pltpu.make_async_copy(
                        t_hbm.at[:, pl.ds(pl.multiple_of(gch * CHUNK, 128),
                                          TAILW)],
                        bufs[par].at[:, pl.ds(0, TAILW)], sems[par]).start()

        def dwait(kd):
            gch = kd * NS + tid
            for par in range(2):
                @pl.when((lax.rem(kd, 2) == par) & (gch < NCH - 1))
                def _(par=par):
                    pltpu.make_async_copy(
                        t_hbm.at[:, pl.ds(0, CHUNK)], bufs[par],
                        sems[par]).wait()

                @pl.when((lax.rem(kd, 2) == par) & (gch == NCH - 1))
                def _(par=par):
                    pltpu.make_async_copy(
                        t_hbm.at[:, pl.ds(0, TAILW)],
                        bufs[par].at[:, pl.ds(0, TAILW)], sems[par]).wait()

        def do_chunk(kd, _):
            gch = kd * NS + tid

            @pl.when(gch <= NCH - 1)
            def _():
                m = lax.fori_loop(0, nv, filt_make(gch), jnp.int32(0))
                dwait(kd)
                for par in range(2):
                    @pl.when((lax.rem(kd, 2) == par) & (m > 0))
                    def _(par=par):
                        proc_chunk(bufs[par], gch, m)

            @pl.when(kd + 2 <= KMAX - 1)
            def _():
                dfetch(kd + 2)
            return 0

        dfetch(jnp.int32(0))
        if KMAX > 1:
            dfetch(jnp.int32(1))

        pltpu.sync_copy(idx_hbm.at[pl.ds(cid * half, half)], idx_v)

        def buck(j, n):
            vec = idx_v[pl.ds(j * L, L)]
            mask = ((vec >> 10) & (NS - 1)) == tid
            pos = n + plsc.cumsum(mask.astype(jnp.int32)) - 1
            plsc.store_scatter(wl_c, [pos], vec, mask=mask)
            bpos = j * L + iota
            plsc.store_scatter(wl_b, [pos], bpos, mask=mask)
            return pos[L - 1] + 1

        n = lax.fori_loop(0, half // L, buck, jnp.int32(0))
        nv = (n + L - 1) // L

        lax.fori_loop(0, KMAX, do_chunk, 0)

        plsc.subcore_barrier()
        pltpu.sync_copy(spm.at[pl.ds(tid * (outw * D), outw * D)], wbsrc)
        iotaD = iota * D

        def transpose_row(d, _):
            for g in range(outw // L):
                val = plsc.load_gather(wbsrc, [iotaD + (g * L * D + d)])
                wb2d[d, pl.ds(g * L, L)] = val
            return 0

        lax.fori_loop(0, D, transpose_row, 0)
        pltpu.sync_copy(
            wb2d,
            out_t_hbm.at[:, pl.ds(pl.multiple_of(cid * half + tid * outw, 128),
                                  outw)])

    return gather_kernel


def kernel(user_id, embedding_table):
    (B,) = user_id.shape
    V, D = embedding_table.shape
    idx = user_id.astype(jnp.int32)
    out_t = _make_gather(V, D, B)(embedding_table.T, idx)
    return out_t.T
